# baseline (speedup 1.0000x reference)
import functools
import math

import jax
import jax.numpy as jnp
from jax import lax
from jax.experimental import pallas as pl
from jax.experimental.pallas import tpu as pltpu
from jax.experimental.pallas import tpu_sc as plsc

TOP_K = 10
K_PARA = 1
LEN_K = K_PARA * TOP_K - 1

_INV_DENOM = [1.0 / math.log2(2.0 + t) for t in range(LEN_K)]

_L = 16
_NW = 32


_NG = 16


def _ds16(base):
    return pl.ds(pl.multiple_of(base, _L), _L)


def _sc_topk_body(y_hbm, idx_hbm, rowbuf, mg, ag, outb, *,
                  n, rows_pw, grp):
    nc = 2
    wid = lax.axis_index("s") * nc + lax.axis_index("c")
    lane = lax.broadcasted_iota(jnp.int32, (_L,), 0)
    nchunk = n // _L
    cpg = nchunk // _NG
    ngrp = rows_pw // grp
    big = jnp.full((_L,), n * _L, jnp.int32)
    neg2 = jnp.full((_L,), -2.0, jnp.float32)

    def fold_max(v):
        s = [v[i] for i in range(_L)]
        while len(s) > 1:
            s = [jnp.maximum(s[i], s[i + len(s) // 2])
                 for i in range(len(s) // 2)]
        return s[0]

    def fold_min_i(v):
        s = [v[i] for i in range(_L)]
        while len(s) > 1:
            s = [jnp.minimum(s[i], s[i + len(s) // 2])
                 for i in range(len(s) // 2)]
        return s[0]

    def scan_group(rbase, g):
        m = jnp.full((_L,), -8.0, jnp.float32)
        a = big
        for k in range(cpg):
            ch = g * cpg + k
            v = rowbuf[_ds16(rbase + ch * _L)]
            gt = v > m
            m = jnp.where(gt, v, m)
            a = jnp.where(gt, ch, a)
        return m, a

    def process_row(r, j):
        rbase = j * n
        dch = (r // _L) * _L
        dl = r - dch
        v = rowbuf[_ds16(rbase + dch)]
        rowbuf[_ds16(rbase + dch)] = jnp.where(lane == dl, -1.0, v)

        for g in range(_NG):
            m, a = scan_group(rbase, g)
            mg[_ds16(g * _L)] = m
            ag[_ds16(g * _L)] = a

        outv = big
        for t in range(LEN_K):
            macc = jnp.full((_L,), -8.0, jnp.float32)
            for g in range(_NG):
                macc = jnp.maximum(macc, mg[_ds16(g * _L)])
            m = fold_max(macc)

            iacc = big
            for g in range(_NG):
                mv = mg[_ds16(g * _L)]
                av = ag[_ds16(g * _L)]
                gi = av * _L + lane
                iacc = jnp.minimum(iacc, jnp.where(mv == m, gi, big))
            ix = fold_min_i(iacc)

            ch16 = (ix >> 4) << 4
            il = ix & (_L - 1)
            w = rowbuf[_ds16(rbase + ch16)]
            rowbuf[_ds16(rbase + ch16)] = jnp.where(lane == il, neg2, w)
            gsel = ix >> 8
            m2, a2 = scan_group(rbase, gsel)
            mg[_ds16(gsel * _L)] = m2
            ag[_ds16(gsel * _L)] = a2

            outv = jnp.where(lane == t, ix, outv)

        outb[_ds16(j * _L)] = outv

    def group(gi, _):
        row0 = wid * rows_pw + gi * grp
        pltpu.sync_copy(y_hbm.at[pl.ds(row0 * n, grp * n)], rowbuf)

        def rows(j, _):
            process_row(row0 + j, j)
            return 0

        lax.fori_loop(0, grp, rows, 0)
        pltpu.sync_copy(outb, idx_hbm.at[pl.ds(row0 * _L, grp * _L)])
        return 0

    lax.fori_loop(0, ngrp, group, 0)


def _sc_topk(y):
    n = y.shape[0]
    rows_pw = n // _NW
    grp = 8
    mesh = plsc.VectorSubcoreMesh(core_axis_name="c", subcore_axis_name="s")
    body = functools.partial(_sc_topk_body, n=n, rows_pw=rows_pw, grp=grp)
    f = pl.kernel(
        body,
        out_type=jax.ShapeDtypeStruct((n * _L,), jnp.int32),
        mesh=mesh,
        scratch_types=[
            pltpu.VMEM((grp * n,), jnp.float32),
            pltpu.VMEM((_NG * _L,), jnp.float32),
            pltpu.VMEM((_NG * _L,), jnp.int32),
            pltpu.VMEM((grp * _L,), jnp.int32),
        ],
    )
    return f(y.reshape(-1)).reshape(n, _L)


def _norm_kernel(o_ref, out_ref):
    o = o_ref[...]
    nrm = jnp.sqrt(jnp.sum(o * o, axis=1, keepdims=True))
    nrm = jnp.where(nrm == 0.0, 1.0, nrm)
    out_ref[...] = o / nrm


def _main_kernel(an_blk_ref, an_full_ref, yidx_ref, out_ref, *, blk, n):
    i = pl.program_id(0)
    an_blk = an_blk_ref[...]
    an_full = an_full_ref[...]

    x = jax.lax.dot_general(
        an_blk, an_full,
        dimension_numbers=(((1,), (1,)), ((), ())),
        preferred_element_type=jnp.float32,
    )
    x = 5.0 * x + 5.0

    col = jax.lax.broadcasted_iota(jnp.int32, (blk, n), 1)
    row = i * blk + jax.lax.broadcasted_iota(jnp.int32, (blk, n), 0)
    diag = col == row

    neg = jnp.float32(-jnp.inf)

    xm = jnp.where(diag, neg, x)
    idcg = jnp.zeros((blk, 1), jnp.float32)
    for t in range(LEN_K):
        m = jnp.max(xm, axis=1, keepdims=True)
        idcg = idcg + (jnp.exp2(m) - 1.0) * _INV_DENOM[t]
        xm = jnp.where(xm == m, neg, xm)

    dcg = jnp.zeros((blk, 1), jnp.float32)
    for t in range(LEN_K):
        sel = col == yidx_ref[:, t:t + 1]
        xg = jnp.max(jnp.where(sel, x, neg), axis=1, keepdims=True)
        dcg = dcg + (jnp.exp2(xg) - 1.0) * _INV_DENOM[t]

    ndcg = dcg / idcg

    @pl.when(i == 0)
    def _():
        out_ref[...] = jnp.zeros((1, 1), jnp.float32)

    out_ref[...] += jnp.sum(ndcg, keepdims=True)


def kernel(output, y_similarity):
    n, d = output.shape

    yidx = _sc_topk(y_similarity)

    a_norm = pl.pallas_call(
        _norm_kernel,
        out_shape=jax.ShapeDtypeStruct((n, d), jnp.float32),
    )(output)

    blk = min(256, n)
    grid = n // blk

    body = functools.partial(_main_kernel, blk=blk, n=n)

    total = pl.pallas_call(
        body,
        grid=(grid,),
        in_specs=[
            pl.BlockSpec((blk, d), lambda i: (i, 0)),
            pl.BlockSpec((n, d), lambda i: (0, 0)),
            pl.BlockSpec((blk, _L), lambda i: (i, 0)),
        ],
        out_specs=pl.BlockSpec((1, 1), lambda i: (0, 0)),
        out_shape=jax.ShapeDtypeStruct((1, 1), jnp.float32),
    )(a_norm, a_norm, yidx)

    return total[0, 0] / n


# speedup vs baseline: 1.0190x; 1.0190x over previous
import functools
import math

import jax
import jax.numpy as jnp
from jax import lax
from jax.experimental import pallas as pl
from jax.experimental.pallas import tpu as pltpu
from jax.experimental.pallas import tpu_sc as plsc

TOP_K = 10
K_PARA = 1
LEN_K = K_PARA * TOP_K - 1

_INV_DENOM = [1.0 / math.log2(2.0 + t) for t in range(LEN_K)]

_L = 16
_NW = 32


_NG = 16


def _ds16(base):
    return pl.ds(pl.multiple_of(base, _L), _L)


def _sc_topk_body(y_hbm, idx_hbm, rowbuf, mg, ag, foldv, foldi, outb, *,
                  n, rows_pw, grp):
    nc = 2
    wid = lax.axis_index("s") * nc + lax.axis_index("c")
    lane = lax.broadcasted_iota(jnp.int32, (_L,), 0)
    nchunk = n // _L
    cpg = nchunk // _NG
    ngrp = rows_pw // grp
    big = jnp.full((_L,), n * _L, jnp.int32)
    neg2 = jnp.full((_L,), -2.0, jnp.float32)

    for s in range(2):
        foldv[_ds16(2 * s * _L + _L)] = jnp.full((_L,), -8.0, jnp.float32)
        foldi[_ds16(2 * s * _L + _L)] = big

    def fold_max(v, s):
        base = 2 * s * _L
        v = jnp.maximum(v, lax.rev(v, (0,)))
        foldv[_ds16(base)] = v
        for sh in (4, 2):
            v = jnp.maximum(v, foldv[pl.ds(base + sh, _L)])
            foldv[_ds16(base)] = v
        return jnp.maximum(v, foldv[pl.ds(base + 1, _L)])[0]

    def fold_min_i(v, s):
        base = 2 * s * _L
        v = jnp.minimum(v, lax.rev(v, (0,)))
        foldi[_ds16(base)] = v
        for sh in (4, 2):
            v = jnp.minimum(v, foldi[pl.ds(base + sh, _L)])
            foldi[_ds16(base)] = v
        return jnp.minimum(v, foldi[pl.ds(base + 1, _L)])[0]

    def scan_group(rbase, g):
        m = jnp.full((_L,), -8.0, jnp.float32)
        a = big
        for k in range(cpg):
            ch = g * cpg + k
            v = rowbuf[_ds16(rbase + ch * _L)]
            gt = v > m
            m = jnp.where(gt, v, m)
            a = jnp.where(gt, ch, a)
        return m, a

    def diag_fix(r, rbase):
        dch = (r // _L) * _L
        dl = r - dch
        v = rowbuf[_ds16(rbase + dch)]
        rowbuf[_ds16(rbase + dch)] = jnp.where(lane == dl, -1.0, v)

    def phase_a(s, rbase):
        for g in range(_NG):
            m, a = scan_group(rbase, g)
            mg[_ds16((s * _NG + g) * _L)] = m
            ag[_ds16((s * _NG + g) * _L)] = a

    def round_step(s, rbase):
        macc = jnp.full((_L,), -8.0, jnp.float32)
        for g in range(_NG):
            macc = jnp.maximum(macc, mg[_ds16((s * _NG + g) * _L)])
        m = fold_max(macc, s)

        iacc = big
        for g in range(_NG):
            mv = mg[_ds16((s * _NG + g) * _L)]
            av = ag[_ds16((s * _NG + g) * _L)]
            gi = av * _L + lane
            iacc = jnp.minimum(iacc, jnp.where(mv == m, gi, big))
        ix = fold_min_i(iacc, s)

        ch16 = (ix >> 4) << 4
        il = ix & (_L - 1)
        w = rowbuf[_ds16(rbase + ch16)]
        rowbuf[_ds16(rbase + ch16)] = jnp.where(lane == il, neg2, w)
        gsel = ix >> 8
        m2, a2 = scan_group(rbase, gsel)
        mg[_ds16((s * _NG + gsel) * _L)] = m2
        ag[_ds16((s * _NG + gsel) * _L)] = a2
        return ix

    def group(gi, _):
        row0 = wid * rows_pw + gi * grp
        pltpu.sync_copy(y_hbm.at[pl.ds(row0 * n, grp * n)], rowbuf)

        def pairs(jp, _):
            j0 = 2 * jp
            j1 = j0 + 1
            b0 = j0 * n
            b1 = j1 * n
            diag_fix(row0 + j0, b0)
            diag_fix(row0 + j1, b1)
            phase_a(0, b0)
            phase_a(1, b1)
            outv0 = big
            outv1 = big
            for t in range(LEN_K):
                ix0 = round_step(0, b0)
                ix1 = round_step(1, b1)
                outv0 = jnp.where(lane == t, ix0, outv0)
                outv1 = jnp.where(lane == t, ix1, outv1)
            outb[_ds16(j0 * _L)] = outv0
            outb[_ds16(j1 * _L)] = outv1
            return 0

        lax.fori_loop(0, grp // 2, pairs, 0)
        pltpu.sync_copy(outb, idx_hbm.at[pl.ds(row0 * _L, grp * _L)])
        return 0

    lax.fori_loop(0, ngrp, group, 0)


def _sc_topk(y):
    n = y.shape[0]
    rows_pw = n // _NW
    grp = 8
    mesh = plsc.VectorSubcoreMesh(core_axis_name="c", subcore_axis_name="s")
    body = functools.partial(_sc_topk_body, n=n, rows_pw=rows_pw, grp=grp)
    f = pl.kernel(
        body,
        out_type=jax.ShapeDtypeStruct((n * _L,), jnp.int32),
        mesh=mesh,
        scratch_types=[
            pltpu.VMEM((grp * n,), jnp.float32),
            pltpu.VMEM((2 * _NG * _L,), jnp.float32),
            pltpu.VMEM((2 * _NG * _L,), jnp.int32),
            pltpu.VMEM((4 * _L,), jnp.float32),
            pltpu.VMEM((4 * _L,), jnp.int32),
            pltpu.VMEM((grp * _L,), jnp.int32),
        ],
    )
    return f(y.reshape(-1)).reshape(n, _L)


def _norm_kernel(o_ref, out_ref):
    o = o_ref[...]
    nrm = jnp.sqrt(jnp.sum(o * o, axis=1, keepdims=True))
    nrm = jnp.where(nrm == 0.0, 1.0, nrm)
    out_ref[...] = o / nrm


def _main_kernel(an_blk_ref, an_full_ref, yidx_ref, out_ref, *, blk, n):
    i = pl.program_id(0)
    an_blk = an_blk_ref[...]
    an_full = an_full_ref[...]

    x = jax.lax.dot_general(
        an_blk, an_full,
        dimension_numbers=(((1,), (1,)), ((), ())),
        preferred_element_type=jnp.float32,
    )
    x = 5.0 * x + 5.0

    col = jax.lax.broadcasted_iota(jnp.int32, (blk, n), 1)
    row = i * blk + jax.lax.broadcasted_iota(jnp.int32, (blk, n), 0)
    diag = col == row

    neg = jnp.float32(-jnp.inf)

    xm = jnp.where(diag, neg, x)
    idcg = jnp.zeros((blk, 1), jnp.float32)
    for t in range(LEN_K):
        m = jnp.max(xm, axis=1, keepdims=True)
        idcg = idcg + (jnp.exp2(m) - 1.0) * _INV_DENOM[t]
        xm = jnp.where(xm == m, neg, xm)

    dcg = jnp.zeros((blk, 1), jnp.float32)
    for t in range(LEN_K):
        sel = col == yidx_ref[:, t:t + 1]
        xg = jnp.max(jnp.where(sel, x, neg), axis=1, keepdims=True)
        dcg = dcg + (jnp.exp2(xg) - 1.0) * _INV_DENOM[t]

    ndcg = dcg / idcg

    @pl.when(i == 0)
    def _():
        out_ref[...] = jnp.zeros((1, 1), jnp.float32)

    out_ref[...] += jnp.sum(ndcg, keepdims=True)


def kernel(output, y_similarity):
    n, d = output.shape

    yidx = _sc_topk(y_similarity)

    a_norm = pl.pallas_call(
        _norm_kernel,
        out_shape=jax.ShapeDtypeStruct((n, d), jnp.float32),
    )(output)

    blk = min(256, n)
    grid = n // blk

    body = functools.partial(_main_kernel, blk=blk, n=n)

    total = pl.pallas_call(
        body,
        grid=(grid,),
        in_specs=[
            pl.BlockSpec((blk, d), lambda i: (i, 0)),
            pl.BlockSpec((n, d), lambda i: (0, 0)),
            pl.BlockSpec((blk, _L), lambda i: (i, 0)),
        ],
        out_specs=pl.BlockSpec((1, 1), lambda i: (0, 0)),
        out_shape=jax.ShapeDtypeStruct((1, 1), jnp.float32),
    )(a_norm, a_norm, yidx)

    return total[0, 0] / n


# speedup vs baseline: 1.1550x; 1.1336x over previous
import functools
import math

import jax
import jax.numpy as jnp
from jax import lax
from jax.experimental import pallas as pl
from jax.experimental.pallas import tpu as pltpu
from jax.experimental.pallas import tpu_sc as plsc

TOP_K = 10
K_PARA = 1
LEN_K = K_PARA * TOP_K - 1

_INV_DENOM = [1.0 / math.log2(2.0 + t) for t in range(LEN_K)]

_L = 16
_NW = 32


_NG = 16


def _ds16(base):
    return pl.ds(pl.multiple_of(base, _L), _L)


def _sc_topk_body(y_hbm, idx_hbm, rowbuf, mg, ag, foldv, foldi, outb, *,
                  n, rows_pw, grp):
    nc = 2
    wid = lax.axis_index("s") * nc + lax.axis_index("c")
    lane = lax.broadcasted_iota(jnp.int32, (_L,), 0)
    nchunk = n // _L
    cpg = nchunk // _NG
    ngrp = rows_pw // grp
    big = jnp.full((_L,), n * _L, jnp.int32)
    neg2 = jnp.full((_L,), -2.0, jnp.float32)

    for s in range(2):
        foldv[_ds16(2 * s * _L + _L)] = jnp.full((_L,), -8.0, jnp.float32)
        foldi[_ds16(2 * s * _L + _L)] = big

    def fold_max(v, s):
        base = 2 * s * _L
        v = jnp.maximum(v, lax.rev(v, (0,)))
        foldv[_ds16(base)] = v
        for sh in (4, 2):
            v = jnp.maximum(v, foldv[pl.ds(base + sh, _L)])
            foldv[_ds16(base)] = v
        return jnp.maximum(v, foldv[pl.ds(base + 1, _L)])[0]

    def fold_min_i(v, s):
        base = 2 * s * _L
        v = jnp.minimum(v, lax.rev(v, (0,)))
        foldi[_ds16(base)] = v
        for sh in (4, 2):
            v = jnp.minimum(v, foldi[pl.ds(base + sh, _L)])
            foldi[_ds16(base)] = v
        return jnp.minimum(v, foldi[pl.ds(base + 1, _L)])[0]

    def scan_group(rbase, g):
        m = jnp.full((_L,), -8.0, jnp.float32)
        a = big
        for k in range(cpg):
            ch = g * cpg + k
            v = rowbuf[_ds16(rbase + ch * _L)]
            gt = v > m
            m = jnp.where(gt, v, m)
            a = jnp.where(gt, ch, a)
        return m, a

    def diag_fix(r, rbase):
        dch = (r // _L) * _L
        dl = r - dch
        v = rowbuf[_ds16(rbase + dch)]
        rowbuf[_ds16(rbase + dch)] = jnp.where(lane == dl, -1.0, v)

    def phase_a(s, rbase):
        for g in range(_NG):
            m, a = scan_group(rbase, g)
            mg[_ds16((s * _NG + g) * _L)] = m
            ag[_ds16((s * _NG + g) * _L)] = a

    def round_step(s, rbase):
        macc = jnp.full((_L,), -8.0, jnp.float32)
        for g in range(_NG):
            macc = jnp.maximum(macc, mg[_ds16((s * _NG + g) * _L)])
        m = fold_max(macc, s)

        iacc = big
        for g in range(_NG):
            mv = mg[_ds16((s * _NG + g) * _L)]
            av = ag[_ds16((s * _NG + g) * _L)]
            gi = av * _L + lane
            iacc = jnp.minimum(iacc, jnp.where(mv == m, gi, big))
        ix = fold_min_i(iacc, s)

        ch16 = (ix >> 4) << 4
        il = ix & (_L - 1)
        w = rowbuf[_ds16(rbase + ch16)]
        rowbuf[_ds16(rbase + ch16)] = jnp.where(lane == il, neg2, w)
        gsel = ix >> 8
        m2, a2 = scan_group(rbase, gsel)
        mg[_ds16((s * _NG + gsel) * _L)] = m2
        ag[_ds16((s * _NG + gsel) * _L)] = a2
        return ix

    def group(gi, _):
        row0 = wid * rows_pw + gi * grp
        pltpu.sync_copy(y_hbm.at[pl.ds(row0 * n, grp * n)], rowbuf)

        def rows(j, _):
            b0 = j * n
            diag_fix(row0 + j, b0)
            phase_a(0, b0)
            outv = big
            for t in range(LEN_K):
                ix = round_step(0, b0)
                outv = jnp.where(lane == t, ix, outv)
            outb[_ds16(j * _L)] = outv
            return 0

        lax.fori_loop(0, grp, rows, 0)
        pltpu.sync_copy(outb, idx_hbm.at[pl.ds(row0 * _L, grp * _L)])
        return 0

    lax.fori_loop(0, ngrp, group, 0)


def _sc_topk(y):
    n = y.shape[0]
    rows_pw = n // _NW
    grp = 8
    mesh = plsc.VectorSubcoreMesh(core_axis_name="c", subcore_axis_name="s")
    body = functools.partial(_sc_topk_body, n=n, rows_pw=rows_pw, grp=grp)
    f = pl.kernel(
        body,
        out_type=jax.ShapeDtypeStruct((n * _L,), jnp.int32),
        mesh=mesh,
        scratch_types=[
            pltpu.VMEM((grp * n,), jnp.float32),
            pltpu.VMEM((2 * _NG * _L,), jnp.float32),
            pltpu.VMEM((2 * _NG * _L,), jnp.int32),
            pltpu.VMEM((4 * _L,), jnp.float32),
            pltpu.VMEM((4 * _L,), jnp.int32),
            pltpu.VMEM((grp * _L,), jnp.int32),
        ],
    )
    return f(y.reshape(-1)).reshape(n, _L)


def _norm_kernel(o_ref, out_ref):
    o = o_ref[...]
    nrm = jnp.sqrt(jnp.sum(o * o, axis=1, keepdims=True))
    nrm = jnp.where(nrm == 0.0, 1.0, nrm)
    out_ref[...] = o / nrm


def _main_kernel(an_blk_ref, an_full_ref, yidx_ref, out_ref, *, blk, n):
    i = pl.program_id(0)
    an_blk = an_blk_ref[...]
    an_full = an_full_ref[...]

    x = jax.lax.dot_general(
        an_blk, an_full,
        dimension_numbers=(((1,), (1,)), ((), ())),
        preferred_element_type=jnp.float32,
    )
    x = 5.0 * x + 5.0

    col = jax.lax.broadcasted_iota(jnp.int32, (blk, n), 1)
    row = i * blk + jax.lax.broadcasted_iota(jnp.int32, (blk, n), 0)
    diag = col == row

    neg = jnp.float32(-jnp.inf)

    xm = jnp.where(diag, neg, x)
    idcg = jnp.zeros((blk, 1), jnp.float32)
    for t in range(LEN_K):
        m = jnp.max(xm, axis=1, keepdims=True)
        idcg = idcg + (jnp.exp2(m) - 1.0) * _INV_DENOM[t]
        xm = jnp.where(xm == m, neg, xm)

    dcg = jnp.zeros((blk, 1), jnp.float32)
    for t in range(LEN_K):
        sel = col == yidx_ref[:, t:t + 1]
        xg = jnp.max(jnp.where(sel, x, neg), axis=1, keepdims=True)
        dcg = dcg + (jnp.exp2(xg) - 1.0) * _INV_DENOM[t]

    ndcg = dcg / idcg

    @pl.when(i == 0)
    def _():
        out_ref[...] = jnp.zeros((1, 1), jnp.float32)

    out_ref[...] += jnp.sum(ndcg, keepdims=True)


def kernel(output, y_similarity):
    n, d = output.shape

    yidx = _sc_topk(y_similarity)

    a_norm = pl.pallas_call(
        _norm_kernel,
        out_shape=jax.ShapeDtypeStruct((n, d), jnp.float32),
    )(output)

    blk = min(256, n)
    grid = n // blk

    body = functools.partial(_main_kernel, blk=blk, n=n)

    total = pl.pallas_call(
        body,
        grid=(grid,),
        in_specs=[
            pl.BlockSpec((blk, d), lambda i: (i, 0)),
            pl.BlockSpec((n, d), lambda i: (0, 0)),
            pl.BlockSpec((blk, _L), lambda i: (i, 0)),
        ],
        out_specs=pl.BlockSpec((1, 1), lambda i: (0, 0)),
        out_shape=jax.ShapeDtypeStruct((1, 1), jnp.float32),
    )(a_norm, a_norm, yidx)

    return total[0, 0] / n


# speedup vs baseline: 1.2220x; 1.0579x over previous
import functools
import math

import jax
import jax.numpy as jnp
from jax import lax
from jax.experimental import pallas as pl
from jax.experimental.pallas import tpu as pltpu
from jax.experimental.pallas import tpu_sc as plsc

TOP_K = 10
K_PARA = 1
LEN_K = K_PARA * TOP_K - 1

_INV_DENOM = [1.0 / math.log2(2.0 + t) for t in range(LEN_K)]

_L = 16
_NW = 32


_NG = 16


def _ds16(base):
    return pl.ds(pl.multiple_of(base, _L), _L)


def _sc_topk_body(y_hbm, idx_hbm, rowbuf, mg, ag, foldv, foldi, outb, *,
                  n, rows_pw, grp):
    nc = 2
    wid = lax.axis_index("s") * nc + lax.axis_index("c")
    lane = lax.broadcasted_iota(jnp.int32, (_L,), 0)
    nchunk = n // _L
    cpg = nchunk // _NG
    ngrp = rows_pw // grp
    big = jnp.full((_L,), n * _L, jnp.int32)
    neg2 = jnp.full((_L,), -2.0, jnp.float32)

    for s in range(2):
        foldv[_ds16(2 * s * _L + _L)] = jnp.full((_L,), -8.0, jnp.float32)
        foldi[_ds16(2 * s * _L + _L)] = big

    def fold_max(v, s):
        base = 2 * s * _L
        v = jnp.maximum(v, lax.rev(v, (0,)))
        foldv[_ds16(base)] = v
        for sh in (4, 2):
            v = jnp.maximum(v, foldv[pl.ds(base + sh, _L)])
            foldv[_ds16(base)] = v
        return jnp.maximum(v, foldv[pl.ds(base + 1, _L)])[0]

    def fold_min_i(v, s):
        base = 2 * s * _L
        v = jnp.minimum(v, lax.rev(v, (0,)))
        foldi[_ds16(base)] = v
        for sh in (4, 2):
            v = jnp.minimum(v, foldi[pl.ds(base + sh, _L)])
            foldi[_ds16(base)] = v
        return jnp.minimum(v, foldi[pl.ds(base + 1, _L)])[0]

    def scan_group(rbase, g):
        items = []
        for k in range(cpg):
            ch = g * cpg + k
            v = rowbuf[_ds16(rbase + ch * _L)]
            items.append((v, jnp.zeros((_L,), jnp.int32) + ch))
        while len(items) > 1:
            nxt = []
            for i in range(0, len(items), 2):
                (lm, la), (rm, ra) = items[i], items[i + 1]
                gt = rm > lm
                nxt.append((jnp.where(gt, rm, lm), jnp.where(gt, ra, la)))
            items = nxt
        return items[0]

    def diag_fix(r, rbase):
        dch = (r // _L) * _L
        dl = r - dch
        v = rowbuf[_ds16(rbase + dch)]
        rowbuf[_ds16(rbase + dch)] = jnp.where(lane == dl, -1.0, v)

    def phase_a(s, rbase):
        for g in range(_NG):
            m, a = scan_group(rbase, g)
            mg[_ds16((s * _NG + g) * _L)] = m
            ag[_ds16((s * _NG + g) * _L)] = a

    def _tree(items, op):
        while len(items) > 1:
            items = [op(items[i], items[i + 1])
                     for i in range(0, len(items), 2)]
        return items[0]

    def round_step(s, rbase):
        mvs = [mg[_ds16((s * _NG + g) * _L)] for g in range(_NG)]
        m = fold_max(_tree(list(mvs), jnp.maximum), s)

        cands = []
        for g in range(_NG):
            av = ag[_ds16((s * _NG + g) * _L)]
            gi = av * _L + lane
            cands.append(jnp.where(mvs[g] == m, gi, big))
        ix = fold_min_i(_tree(cands, jnp.minimum), s)

        ch16 = (ix >> 4) << 4
        il = ix & (_L - 1)
        w = rowbuf[_ds16(rbase + ch16)]
        rowbuf[_ds16(rbase + ch16)] = jnp.where(lane == il, neg2, w)
        gsel = ix >> 8
        m2, a2 = scan_group(rbase, gsel)
        mg[_ds16((s * _NG + gsel) * _L)] = m2
        ag[_ds16((s * _NG + gsel) * _L)] = a2
        return ix

    def group(gi, _):
        row0 = wid * rows_pw + gi * grp
        pltpu.sync_copy(y_hbm.at[pl.ds(row0 * n, grp * n)], rowbuf)

        def rows(j, _):
            b0 = j * n
            diag_fix(row0 + j, b0)
            phase_a(0, b0)
            outv = big
            for t in range(LEN_K):
                ix = round_step(0, b0)
                outv = jnp.where(lane == t, ix, outv)
            outb[_ds16(j * _L)] = outv
            return 0

        lax.fori_loop(0, grp, rows, 0)
        pltpu.sync_copy(outb, idx_hbm.at[pl.ds(row0 * _L, grp * _L)])
        return 0

    lax.fori_loop(0, ngrp, group, 0)


def _sc_topk(y):
    n = y.shape[0]
    rows_pw = n // _NW
    grp = 8
    mesh = plsc.VectorSubcoreMesh(core_axis_name="c", subcore_axis_name="s")
    body = functools.partial(_sc_topk_body, n=n, rows_pw=rows_pw, grp=grp)
    f = pl.kernel(
        body,
        out_type=jax.ShapeDtypeStruct((n * _L,), jnp.int32),
        mesh=mesh,
        scratch_types=[
            pltpu.VMEM((grp * n,), jnp.float32),
            pltpu.VMEM((2 * _NG * _L,), jnp.float32),
            pltpu.VMEM((2 * _NG * _L,), jnp.int32),
            pltpu.VMEM((4 * _L,), jnp.float32),
            pltpu.VMEM((4 * _L,), jnp.int32),
            pltpu.VMEM((grp * _L,), jnp.int32),
        ],
    )
    return f(y.reshape(-1)).reshape(n, _L)


def _norm_kernel(o_ref, out_ref):
    o = o_ref[...]
    nrm = jnp.sqrt(jnp.sum(o * o, axis=1, keepdims=True))
    nrm = jnp.where(nrm == 0.0, 1.0, nrm)
    out_ref[...] = o / nrm


def _main_kernel(an_blk_ref, an_full_ref, yidx_ref, out_ref, *, blk, n):
    i = pl.program_id(0)
    an_blk = an_blk_ref[...]
    an_full = an_full_ref[...]

    x = jax.lax.dot_general(
        an_blk, an_full,
        dimension_numbers=(((1,), (1,)), ((), ())),
        preferred_element_type=jnp.float32,
    )
    x = 5.0 * x + 5.0

    col = jax.lax.broadcasted_iota(jnp.int32, (blk, n), 1)
    row = i * blk + jax.lax.broadcasted_iota(jnp.int32, (blk, n), 0)
    diag = col == row

    neg = jnp.float32(-jnp.inf)

    xm = jnp.where(diag, neg, x)
    idcg = jnp.zeros((blk, 1), jnp.float32)
    for t in range(LEN_K):
        m = jnp.max(xm, axis=1, keepdims=True)
        idcg = idcg + (jnp.exp2(m) - 1.0) * _INV_DENOM[t]
        xm = jnp.where(xm == m, neg, xm)

    dcg = jnp.zeros((blk, 1), jnp.float32)
    for t in range(LEN_K):
        sel = col == yidx_ref[:, t:t + 1]
        xg = jnp.max(jnp.where(sel, x, neg), axis=1, keepdims=True)
        dcg = dcg + (jnp.exp2(xg) - 1.0) * _INV_DENOM[t]

    ndcg = dcg / idcg

    @pl.when(i == 0)
    def _():
        out_ref[...] = jnp.zeros((1, 1), jnp.float32)

    out_ref[...] += jnp.sum(ndcg, keepdims=True)


def kernel(output, y_similarity):
    n, d = output.shape

    yidx = _sc_topk(y_similarity)

    a_norm = pl.pallas_call(
        _norm_kernel,
        out_shape=jax.ShapeDtypeStruct((n, d), jnp.float32),
    )(output)

    blk = min(256, n)
    grid = n // blk

    body = functools.partial(_main_kernel, blk=blk, n=n)

    total = pl.pallas_call(
        body,
        grid=(grid,),
        in_specs=[
            pl.BlockSpec((blk, d), lambda i: (i, 0)),
            pl.BlockSpec((n, d), lambda i: (0, 0)),
            pl.BlockSpec((blk, _L), lambda i: (i, 0)),
        ],
        out_specs=pl.BlockSpec((1, 1), lambda i: (0, 0)),
        out_shape=jax.ShapeDtypeStruct((1, 1), jnp.float32),
    )(a_norm, a_norm, yidx)

    return total[0, 0] / n


# speedup vs baseline: 1.6772x; 1.3725x over previous
import functools
import math

import jax
import jax.numpy as jnp
from jax import lax
from jax.experimental import pallas as pl
from jax.experimental.pallas import tpu as pltpu
from jax.experimental.pallas import tpu_sc as plsc

TOP_K = 10
K_PARA = 1
LEN_K = K_PARA * TOP_K - 1

_INV_DENOM = [1.0 / math.log2(2.0 + t) for t in range(LEN_K)]

_L = 16
_NW = 32
_CW = 128
_NCH = 32


def _ymax_kernel(y_ref, out_ref, *, blk, n):
    i = pl.program_id(0)
    y = y_ref[...]
    rowg = i * blk + lax.broadcasted_iota(jnp.int32, (blk, _CW), 0)
    cols = []
    for ci in range(n // _CW):
        col = ci * _CW + lax.broadcasted_iota(jnp.int32, (blk, _CW), 1)
        ysl = jnp.where(col == rowg, -1.0, y[:, ci * _CW:(ci + 1) * _CW])
        cols.append(jnp.max(ysl, axis=1, keepdims=True))
    out_ref[...] = jnp.concatenate(cols, axis=1)


def _ds16(base):
    return pl.ds(pl.multiple_of(base, _L), _L)


def _tree(items, op):
    while len(items) > 1:
        items = [op(items[i], items[i + 1])
                 for i in range(0, len(items), 2)]
    return items[0]


def _sc_topk_body(y_hbm, ymax_hbm, idx_hbm, rowbuf, mvbuf, foldv, foldi,
                  outb, *, n, rows_pw, grp):
    nc = 2
    wid = lax.axis_index("s") * nc + lax.axis_index("c")
    lane = lax.broadcasted_iota(jnp.int32, (_L,), 0)
    ngrp = rows_pw // grp
    big = jnp.full((_L,), n * _L, jnp.int32)
    neg2 = jnp.full((_L,), -2.0, jnp.float32)

    foldv[_ds16(_L)] = jnp.full((_L,), -8.0, jnp.float32)
    foldi[_ds16(_L)] = big

    def fold_max(v):
        v = jnp.maximum(v, lax.rev(v, (0,)))
        foldv[_ds16(0)] = v
        for sh in (4, 2):
            v = jnp.maximum(v, foldv[pl.ds(sh, _L)])
            foldv[_ds16(0)] = v
        return jnp.maximum(v, foldv[pl.ds(1, _L)])[0]

    def fold_min_i(v):
        v = jnp.minimum(v, lax.rev(v, (0,)))
        foldi[_ds16(0)] = v
        for sh in (4, 2):
            v = jnp.minimum(v, foldi[pl.ds(sh, _L)])
            foldi[_ds16(0)] = v
        return jnp.minimum(v, foldi[pl.ds(1, _L)])[0]

    def diag_fix(r, rbase):
        dch = (r // _L) * _L
        dl = r - dch
        v = rowbuf[_ds16(rbase + dch)]
        rowbuf[_ds16(rbase + dch)] = jnp.where(lane == dl, -1.0, v)

    def group(gi, _):
        row0 = wid * rows_pw + gi * grp
        pltpu.sync_copy(y_hbm.at[pl.ds(row0 * n, grp * n)], rowbuf)
        pltpu.sync_copy(ymax_hbm.at[pl.ds(row0 * _NCH, grp * _NCH)], mvbuf)

        def rows(j, _):
            b0 = j * n
            mb = j * _NCH
            diag_fix(row0 + j, b0)
            outv = big
            for t in range(LEN_K):
                mv0 = mvbuf[_ds16(mb)]
                mv1 = mvbuf[_ds16(mb + _L)]
                m = fold_max(jnp.maximum(mv0, mv1))
                c0 = jnp.where(mv0 == m, lane, big)
                c1 = jnp.where(mv1 == m, lane + _L, big)
                c = fold_min_i(jnp.minimum(c0, c1))
                cb = b0 + c * _CW
                cands = []
                for k in range(_CW // _L):
                    v = rowbuf[_ds16(cb + k * _L)]
                    gidx = c * _CW + k * _L + lane
                    cands.append(jnp.where(v == m, gidx, big))
                ix = fold_min_i(_tree(cands, jnp.minimum))
                outv = jnp.where(lane == t, ix, outv)
                ch16 = (ix >> 4) << 4
                il = ix & (_L - 1)
                w = rowbuf[_ds16(b0 + ch16)]
                rowbuf[_ds16(b0 + ch16)] = jnp.where(lane == il, neg2, w)
                vs = [rowbuf[_ds16(cb + k * _L)] for k in range(_CW // _L)]
                m2 = fold_max(_tree(vs, jnp.maximum))
                hi = c >> 4
                off = mb + hi * _L
                old = mvbuf[_ds16(off)]
                mvbuf[_ds16(off)] = jnp.where(lane == (c & (_L - 1)), m2, old)
            outb[_ds16(j * _L)] = outv
            return 0

        lax.fori_loop(0, grp, rows, 0)
        pltpu.sync_copy(outb, idx_hbm.at[pl.ds(row0 * _L, grp * _L)])
        return 0

    lax.fori_loop(0, ngrp, group, 0)


def _sc_topk(y, ymax):
    n = y.shape[0]
    rows_pw = n // _NW
    grp = 8
    mesh = plsc.VectorSubcoreMesh(core_axis_name="c", subcore_axis_name="s")
    body = functools.partial(_sc_topk_body, n=n, rows_pw=rows_pw, grp=grp)
    f = pl.kernel(
        body,
        out_type=jax.ShapeDtypeStruct((n * _L,), jnp.int32),
        mesh=mesh,
        scratch_types=[
            pltpu.VMEM((grp * n,), jnp.float32),
            pltpu.VMEM((grp * _NCH,), jnp.float32),
            pltpu.VMEM((2 * _L,), jnp.float32),
            pltpu.VMEM((2 * _L,), jnp.int32),
            pltpu.VMEM((grp * _L,), jnp.int32),
        ],
    )
    return f(y.reshape(-1), ymax.reshape(-1)).reshape(n, _L)


def _norm_kernel(o_ref, out_ref):
    o = o_ref[...]
    nrm = jnp.sqrt(jnp.sum(o * o, axis=1, keepdims=True))
    nrm = jnp.where(nrm == 0.0, 1.0, nrm)
    out_ref[...] = o / nrm


def _main_kernel(an_blk_ref, an_full_ref, yidx_ref, out_ref, *, blk, n):
    i = pl.program_id(0)
    an_blk = an_blk_ref[...]
    an_full = an_full_ref[...]

    x = jax.lax.dot_general(
        an_blk, an_full,
        dimension_numbers=(((1,), (1,)), ((), ())),
        preferred_element_type=jnp.float32,
    )
    x = 5.0 * x + 5.0

    col = jax.lax.broadcasted_iota(jnp.int32, (blk, n), 1)
    row = i * blk + jax.lax.broadcasted_iota(jnp.int32, (blk, n), 0)
    diag = col == row

    neg = jnp.float32(-jnp.inf)

    xm = jnp.where(diag, neg, x)
    idcg = jnp.zeros((blk, 1), jnp.float32)
    for t in range(LEN_K):
        m = jnp.max(xm, axis=1, keepdims=True)
        idcg = idcg + (jnp.exp2(m) - 1.0) * _INV_DENOM[t]
        xm = jnp.where(xm == m, neg, xm)

    dcg = jnp.zeros((blk, 1), jnp.float32)
    for t in range(LEN_K):
        sel = col == yidx_ref[:, t:t + 1]
        xg = jnp.max(jnp.where(sel, x, neg), axis=1, keepdims=True)
        dcg = dcg + (jnp.exp2(xg) - 1.0) * _INV_DENOM[t]

    ndcg = dcg / idcg

    @pl.when(i == 0)
    def _():
        out_ref[...] = jnp.zeros((1, 1), jnp.float32)

    out_ref[...] += jnp.sum(ndcg, keepdims=True)


def kernel(output, y_similarity):
    n, d = output.shape

    blk = min(256, n)
    grid = n // blk

    ymax = pl.pallas_call(
        functools.partial(_ymax_kernel, blk=blk, n=n),
        grid=(grid,),
        in_specs=[pl.BlockSpec((blk, n), lambda i: (i, 0))],
        out_specs=pl.BlockSpec((blk, _NCH), lambda i: (i, 0)),
        out_shape=jax.ShapeDtypeStruct((n, _NCH), jnp.float32),
    )(y_similarity)

    yidx = _sc_topk(y_similarity, ymax)

    a_norm = pl.pallas_call(
        _norm_kernel,
        out_shape=jax.ShapeDtypeStruct((n, d), jnp.float32),
    )(output)

    body = functools.partial(_main_kernel, blk=blk, n=n)

    total = pl.pallas_call(
        body,
        grid=(grid,),
        in_specs=[
            pl.BlockSpec((blk, d), lambda i: (i, 0)),
            pl.BlockSpec((n, d), lambda i: (0, 0)),
            pl.BlockSpec((blk, _L), lambda i: (i, 0)),
        ],
        out_specs=pl.BlockSpec((1, 1), lambda i: (0, 0)),
        out_shape=jax.ShapeDtypeStruct((1, 1), jnp.float32),
    )(a_norm, a_norm, yidx)

    return total[0, 0] / n


# speedup vs baseline: 2.0490x; 1.2217x over previous
import functools
import math

import jax
import jax.numpy as jnp
from jax import lax
from jax.experimental import pallas as pl
from jax.experimental.pallas import tpu as pltpu
from jax.experimental.pallas import tpu_sc as plsc

TOP_K = 10
K_PARA = 1
LEN_K = K_PARA * TOP_K - 1

_INV_DENOM = [1.0 / math.log2(2.0 + t) for t in range(LEN_K)]

_L = 16
_NW = 32
_CW = 128
_NCH = 32


def _ymax_kernel(y_ref, out_ref, *, blk, n):
    i = pl.program_id(0)
    y = y_ref[...]
    rowg = i * blk + lax.broadcasted_iota(jnp.int32, (blk, _CW), 0)
    cols = []
    for ci in range(n // _CW):
        col = ci * _CW + lax.broadcasted_iota(jnp.int32, (blk, _CW), 1)
        ysl = jnp.where(col == rowg, -1.0, y[:, ci * _CW:(ci + 1) * _CW])
        cols.append(jnp.max(ysl, axis=1, keepdims=True))
    out_ref[...] = jnp.concatenate(cols, axis=1)


def _ds16(base):
    return pl.ds(pl.multiple_of(base, _L), _L)


def _tree(items, op):
    while len(items) > 1:
        items = [op(items[i], items[i + 1])
                 for i in range(0, len(items), 2)]
    return items[0]


def _sc_topk_body(y_hbm, ymax_hbm, idx_hbm, rowbuf, mvbuf, foldv, foldi,
                  outb, sems, *, n, rows_pw, grp):
    nc = 2
    wid = lax.axis_index("s") * nc + lax.axis_index("c")
    lane = lax.broadcasted_iota(jnp.int32, (_L,), 0)
    ngrp = rows_pw // grp
    big = jnp.full((_L,), n * _L, jnp.int32)
    neg2 = jnp.full((_L,), -2.0, jnp.float32)

    foldv[_ds16(_L)] = jnp.full((_L,), -8.0, jnp.float32)
    foldi[_ds16(_L)] = big

    def fold_max(v):
        v = jnp.maximum(v, lax.rev(v, (0,)))
        foldv[_ds16(0)] = v
        for sh in (4, 2):
            v = jnp.maximum(v, foldv[pl.ds(sh, _L)])
            foldv[_ds16(0)] = v
        return jnp.maximum(v, foldv[pl.ds(1, _L)])[0]

    def fold_min_i(v):
        v = jnp.minimum(v, lax.rev(v, (0,)))
        foldi[_ds16(0)] = v
        for sh in (4, 2):
            v = jnp.minimum(v, foldi[pl.ds(sh, _L)])
            foldi[_ds16(0)] = v
        return jnp.minimum(v, foldi[pl.ds(1, _L)])[0]

    def start(gi, b):
        row0 = wid * rows_pw + gi * grp
        sem = sems[b]
        pltpu.async_copy(y_hbm.at[pl.ds(row0, grp)], rowbuf.at[b], sem)
        pltpu.async_copy(ymax_hbm.at[pl.ds(row0, grp)], mvbuf.at[b], sem)

    def drain(b):
        sem = sems[b]
        pltpu.make_async_copy(
            y_hbm.at[pl.ds(0, grp)], rowbuf.at[b], sem).wait()
        pltpu.make_async_copy(
            ymax_hbm.at[pl.ds(0, grp)], mvbuf.at[b], sem).wait()

    def process(gi, b):
        row0 = wid * rows_pw + gi * grp
        rb = rowbuf.at[b]
        mvb = mvbuf.at[b]

        def rows(j, _):
            r = row0 + j
            dch = (r // _L) * _L
            dl = r - dch
            dv = rb[j, _ds16(dch)]
            rb[j, _ds16(dch)] = jnp.where(lane == dl, -1.0, dv)

            outv = big
            for t in range(LEN_K):
                mv0 = mvb[j, _ds16(0)]
                mv1 = mvb[j, _ds16(_L)]
                m = fold_max(jnp.maximum(mv0, mv1))
                c0 = jnp.where(mv0 == m, lane, big)
                c1 = jnp.where(mv1 == m, lane + _L, big)
                c = fold_min_i(jnp.minimum(c0, c1))
                cb = c * _CW
                cands = []
                for k in range(_CW // _L):
                    v = rb[j, _ds16(cb + k * _L)]
                    gidx = cb + k * _L + lane
                    cands.append(jnp.where(v == m, gidx, big))
                ix = fold_min_i(_tree(cands, jnp.minimum))
                outv = jnp.where(lane == t, ix, outv)
                ch16 = (ix >> 4) << 4
                il = ix & (_L - 1)
                w = rb[j, _ds16(ch16)]
                rb[j, _ds16(ch16)] = jnp.where(lane == il, neg2, w)
                vs = [rb[j, _ds16(cb + k * _L)] for k in range(_CW // _L)]
                m2 = fold_max(_tree(vs, jnp.maximum))
                hi = c >> 4
                old = mvb[j, _ds16(hi * _L)]
                mvb[j, _ds16(hi * _L)] = jnp.where(
                    lane == (c & (_L - 1)), m2, old)
            outb[j, _ds16(0)] = outv
            return 0

        lax.fori_loop(0, grp, rows, 0)
        pltpu.sync_copy(outb, idx_hbm.at[pl.ds(row0, grp)])

    start(0, 0)

    def outer(h, _):
        for b in range(2):
            gi = h * 2 + b
            nxt = gi + 1

            @pl.when(nxt < ngrp)
            def _():
                start(nxt, 1 - b)

            drain(b)
            process(gi, b)
        return 0

    lax.fori_loop(0, ngrp // 2, outer, 0)


def _sc_topk(y, ymax):
    n = y.shape[0]
    rows_pw = n // _NW
    grp = 8
    mesh = plsc.VectorSubcoreMesh(core_axis_name="c", subcore_axis_name="s")
    body = functools.partial(_sc_topk_body, n=n, rows_pw=rows_pw, grp=grp)
    f = pl.kernel(
        body,
        out_type=jax.ShapeDtypeStruct((n, _L), jnp.int32),
        mesh=mesh,
        scratch_types=[
            pltpu.VMEM((2, grp, n), jnp.float32),
            pltpu.VMEM((2, grp, _NCH), jnp.float32),
            pltpu.VMEM((2 * _L,), jnp.float32),
            pltpu.VMEM((2 * _L,), jnp.int32),
            pltpu.VMEM((grp, _L), jnp.int32),
            [pltpu.SemaphoreType.DMA, pltpu.SemaphoreType.DMA],
        ],
    )
    return f(y, ymax)


def _norm_kernel(o_ref, out_ref):
    o = o_ref[...]
    nrm = jnp.sqrt(jnp.sum(o * o, axis=1, keepdims=True))
    nrm = jnp.where(nrm == 0.0, 1.0, nrm)
    out_ref[...] = o / nrm


def _main_kernel(an_blk_ref, an_full_ref, yidx_ref, out_ref, *, blk, n):
    i = pl.program_id(0)
    an_blk = an_blk_ref[...]
    an_full = an_full_ref[...]

    x = jax.lax.dot_general(
        an_blk, an_full,
        dimension_numbers=(((1,), (1,)), ((), ())),
        preferred_element_type=jnp.float32,
    )
    x = 5.0 * x + 5.0

    col = jax.lax.broadcasted_iota(jnp.int32, (blk, n), 1)
    row = i * blk + jax.lax.broadcasted_iota(jnp.int32, (blk, n), 0)
    diag = col == row

    neg = jnp.float32(-jnp.inf)

    xm = jnp.where(diag, neg, x)
    idcg = jnp.zeros((blk, 1), jnp.float32)
    for t in range(LEN_K):
        m = jnp.max(xm, axis=1, keepdims=True)
        idcg = idcg + (jnp.exp2(m) - 1.0) * _INV_DENOM[t]
        xm = jnp.where(xm == m, neg, xm)

    dcg = jnp.zeros((blk, 1), jnp.float32)
    for t in range(LEN_K):
        sel = col == yidx_ref[:, t:t + 1]
        xg = jnp.max(jnp.where(sel, x, neg), axis=1, keepdims=True)
        dcg = dcg + (jnp.exp2(xg) - 1.0) * _INV_DENOM[t]

    ndcg = dcg / idcg

    @pl.when(i == 0)
    def _():
        out_ref[...] = jnp.zeros((1, 1), jnp.float32)

    out_ref[...] += jnp.sum(ndcg, keepdims=True)


def kernel(output, y_similarity):
    n, d = output.shape

    blk = min(256, n)
    grid = n // blk

    ymax = pl.pallas_call(
        functools.partial(_ymax_kernel, blk=blk, n=n),
        grid=(grid,),
        in_specs=[pl.BlockSpec((blk, n), lambda i: (i, 0))],
        out_specs=pl.BlockSpec((blk, _NCH), lambda i: (i, 0)),
        out_shape=jax.ShapeDtypeStruct((n, _NCH), jnp.float32),
    )(y_similarity)

    yidx = _sc_topk(y_similarity, ymax)

    a_norm = pl.pallas_call(
        _norm_kernel,
        out_shape=jax.ShapeDtypeStruct((n, d), jnp.float32),
    )(output)

    body = functools.partial(_main_kernel, blk=blk, n=n)

    total = pl.pallas_call(
        body,
        grid=(grid,),
        in_specs=[
            pl.BlockSpec((blk, d), lambda i: (i, 0)),
            pl.BlockSpec((n, d), lambda i: (0, 0)),
            pl.BlockSpec((blk, _L), lambda i: (i, 0)),
        ],
        out_specs=pl.BlockSpec((1, 1), lambda i: (0, 0)),
        out_shape=jax.ShapeDtypeStruct((1, 1), jnp.float32),
    )(a_norm, a_norm, yidx)

    return total[0, 0] / n


# speedup vs baseline: 2.5830x; 1.2606x over previous
import functools
import math

import jax
import jax.numpy as jnp
from jax import lax
from jax.experimental import pallas as pl
from jax.experimental.pallas import tpu as pltpu
from jax.experimental.pallas import tpu_sc as plsc

TOP_K = 10
K_PARA = 1
LEN_K = K_PARA * TOP_K - 1

_INV_DENOM = [1.0 / math.log2(2.0 + t) for t in range(LEN_K)]

_L = 16
_NW = 32
_CW = 128
_NCH = 32


def _ymax_kernel(y_ref, out_ref, *, blk, n):
    i = pl.program_id(0)
    y = y_ref[...]
    rowg = i * blk + lax.broadcasted_iota(jnp.int32, (blk, _CW), 0)
    cols = []
    for ci in range(n // _CW):
        col = ci * _CW + lax.broadcasted_iota(jnp.int32, (blk, _CW), 1)
        ysl = jnp.where(col == rowg, -1.0, y[:, ci * _CW:(ci + 1) * _CW])
        cols.append(jnp.max(ysl, axis=1, keepdims=True))
    out_ref[...] = jnp.concatenate(cols, axis=1)


def _ds16(base):
    return pl.ds(pl.multiple_of(base, _L), _L)


def _tree(items, op):
    while len(items) > 1:
        items = [op(items[i], items[i + 1])
                 for i in range(0, len(items), 2)]
    return items[0]


def _sc_topk_body(y_hbm, ymax_hbm, idx_hbm, rowbuf, mvbuf, foldv, foldi,
                  outb, sems, *, n, rows_pw, grp):
    nc = 2
    wid = lax.axis_index("s") * nc + lax.axis_index("c")
    lane = lax.broadcasted_iota(jnp.int32, (_L,), 0)
    ngrp = rows_pw // grp
    big = jnp.full((_L,), n * _L, jnp.int32)
    neg2 = jnp.full((_L,), -2.0, jnp.float32)

    foldv[_ds16(_L)] = jnp.full((_L,), -8.0, jnp.float32)
    foldi[_ds16(_L)] = big

    def fold_max(v):
        v = jnp.maximum(v, lax.rev(v, (0,)))
        foldv[_ds16(0)] = v
        for sh in (4, 2):
            v = jnp.maximum(v, foldv[pl.ds(sh, _L)])
            foldv[_ds16(0)] = v
        return jnp.maximum(v, foldv[pl.ds(1, _L)])[0]

    def fold_min_i(v):
        v = jnp.minimum(v, lax.rev(v, (0,)))
        foldi[_ds16(0)] = v
        for sh in (4, 2):
            v = jnp.minimum(v, foldi[pl.ds(sh, _L)])
            foldi[_ds16(0)] = v
        return jnp.minimum(v, foldi[pl.ds(1, _L)])[0]

    def start(gi, b):
        row0 = wid * rows_pw + gi * grp
        sem = sems[b]
        pltpu.async_copy(y_hbm.at[pl.ds(row0, grp)], rowbuf.at[b], sem)
        pltpu.async_copy(ymax_hbm.at[pl.ds(row0, grp)], mvbuf.at[b], sem)

    def drain(b):
        sem = sems[b]
        pltpu.make_async_copy(
            y_hbm.at[pl.ds(0, grp)], rowbuf.at[b], sem).wait()
        pltpu.make_async_copy(
            ymax_hbm.at[pl.ds(0, grp)], mvbuf.at[b], sem).wait()

    def process(gi, b):
        row0 = wid * rows_pw + gi * grp
        rb = rowbuf.at[b]
        mvb = mvbuf.at[b]

        def rows(j, _):
            r = row0 + j
            dch = (r // _L) * _L
            dl = r - dch
            dv = rb[j, _ds16(dch)]
            rb[j, _ds16(dch)] = jnp.where(lane == dl, -1.0, dv)

            outv = big
            for t in range(LEN_K):
                mv0 = mvb[j, _ds16(0)]
                mv1 = mvb[j, _ds16(_L)]
                m = fold_max(jnp.maximum(mv0, mv1))
                c0 = jnp.where(mv0 == m, lane, big)
                c1 = jnp.where(mv1 == m, lane + _L, big)
                c = fold_min_i(jnp.minimum(c0, c1))
                cb = c * _CW
                cands = []
                for k in range(_CW // _L):
                    v = rb[j, _ds16(cb + k * _L)]
                    gidx = cb + k * _L + lane
                    cands.append(jnp.where(v == m, gidx, big))
                ix = fold_min_i(_tree(cands, jnp.minimum))
                outv = jnp.where(lane == t, ix, outv)
                ch16 = (ix >> 4) << 4
                il = ix & (_L - 1)
                w = rb[j, _ds16(ch16)]
                rb[j, _ds16(ch16)] = jnp.where(lane == il, neg2, w)
                vs = [rb[j, _ds16(cb + k * _L)] for k in range(_CW // _L)]
                m2 = fold_max(_tree(vs, jnp.maximum))
                hi = c >> 4
                old = mvb[j, _ds16(hi * _L)]
                mvb[j, _ds16(hi * _L)] = jnp.where(
                    lane == (c & (_L - 1)), m2, old)
            outb[j, _ds16(0)] = outv
            return 0

        lax.fori_loop(0, grp, rows, 0)
        pltpu.sync_copy(outb, idx_hbm.at[pl.ds(row0, grp)])

    start(0, 0)

    def outer(h, _):
        for b in range(2):
            gi = h * 2 + b
            nxt = gi + 1

            @pl.when(nxt < ngrp)
            def _():
                start(nxt, 1 - b)

            drain(b)
            process(gi, b)
        return 0

    lax.fori_loop(0, ngrp // 2, outer, 0)


def _sc_topk(y, ymax):
    n = y.shape[0]
    rows_pw = n // _NW
    grp = 8
    mesh = plsc.VectorSubcoreMesh(core_axis_name="c", subcore_axis_name="s")
    body = functools.partial(_sc_topk_body, n=n, rows_pw=rows_pw, grp=grp)
    f = pl.kernel(
        body,
        out_type=jax.ShapeDtypeStruct((n, _L), jnp.int32),
        mesh=mesh,
        scratch_types=[
            pltpu.VMEM((2, grp, n), jnp.float32),
            pltpu.VMEM((2, grp, _NCH), jnp.float32),
            pltpu.VMEM((2 * _L,), jnp.float32),
            pltpu.VMEM((2 * _L,), jnp.int32),
            pltpu.VMEM((grp, _L), jnp.int32),
            [pltpu.SemaphoreType.DMA, pltpu.SemaphoreType.DMA],
        ],
    )
    return f(y, ymax)


def _norm_kernel(o_ref, out_ref):
    o = o_ref[...]
    nrm = jnp.sqrt(jnp.sum(o * o, axis=1, keepdims=True))
    nrm = jnp.where(nrm == 0.0, 1.0, nrm)
    out_ref[...] = o / nrm


def _mat_kernel(an_blk_ref, an_full_ref, x_ref, idcg_ref, *, blk, n):
    i = pl.program_id(0)
    an_blk = an_blk_ref[...]
    an_full = an_full_ref[...]

    x = jax.lax.dot_general(
        an_blk, an_full,
        dimension_numbers=(((1,), (1,)), ((), ())),
        preferred_element_type=jnp.float32,
    )
    x = 5.0 * x + 5.0
    x_ref[...] = x

    col = jax.lax.broadcasted_iota(jnp.int32, (blk, n), 1)
    row = i * blk + jax.lax.broadcasted_iota(jnp.int32, (blk, n), 0)
    diag = col == row

    neg = jnp.float32(-jnp.inf)

    xm = jnp.where(diag, neg, x)
    idcg = jnp.zeros((blk, 1), jnp.float32)
    for t in range(LEN_K):
        m = jnp.max(xm, axis=1, keepdims=True)
        idcg = idcg + (jnp.exp2(m) - 1.0) * _INV_DENOM[t]
        xm = jnp.where(xm == m, neg, xm)
    idcg_ref[...] = idcg


def _dcg_kernel(x_ref, yidx_ref, idcg_ref, out_ref, *, blk, n):
    i = pl.program_id(0)
    x = x_ref[...]
    col = jax.lax.broadcasted_iota(jnp.int32, (blk, n), 1)
    neg = jnp.float32(-jnp.inf)

    dcg = jnp.zeros((blk, 1), jnp.float32)
    for t in range(LEN_K):
        sel = col == yidx_ref[:, t:t + 1]
        xg = jnp.max(jnp.where(sel, x, neg), axis=1, keepdims=True)
        dcg = dcg + (jnp.exp2(xg) - 1.0) * _INV_DENOM[t]

    ndcg = dcg / idcg_ref[...]

    @pl.when(i == 0)
    def _():
        out_ref[...] = jnp.zeros((1, 1), jnp.float32)

    out_ref[...] += jnp.sum(ndcg, keepdims=True)


def kernel(output, y_similarity):
    n, d = output.shape

    blk = min(256, n)
    grid = n // blk

    ymax = pl.pallas_call(
        functools.partial(_ymax_kernel, blk=blk, n=n),
        grid=(grid,),
        in_specs=[pl.BlockSpec((blk, n), lambda i: (i, 0))],
        out_specs=pl.BlockSpec((blk, _NCH), lambda i: (i, 0)),
        out_shape=jax.ShapeDtypeStruct((n, _NCH), jnp.float32),
    )(y_similarity)

    yidx = _sc_topk(y_similarity, ymax)

    a_norm = pl.pallas_call(
        _norm_kernel,
        out_shape=jax.ShapeDtypeStruct((n, d), jnp.float32),
    )(output)

    x_sim, idcg = pl.pallas_call(
        functools.partial(_mat_kernel, blk=blk, n=n),
        grid=(grid,),
        in_specs=[
            pl.BlockSpec((blk, d), lambda i: (i, 0)),
            pl.BlockSpec((n, d), lambda i: (0, 0)),
        ],
        out_specs=[
            pl.BlockSpec((blk, n), lambda i: (i, 0)),
            pl.BlockSpec((blk, 1), lambda i: (i, 0)),
        ],
        out_shape=[
            jax.ShapeDtypeStruct((n, n), jnp.float32),
            jax.ShapeDtypeStruct((n, 1), jnp.float32),
        ],
    )(a_norm, a_norm)

    total = pl.pallas_call(
        functools.partial(_dcg_kernel, blk=blk, n=n),
        grid=(grid,),
        in_specs=[
            pl.BlockSpec((blk, n), lambda i: (i, 0)),
            pl.BlockSpec((blk, _L), lambda i: (i, 0)),
            pl.BlockSpec((blk, 1), lambda i: (i, 0)),
        ],
        out_specs=pl.BlockSpec((1, 1), lambda i: (0, 0)),
        out_shape=jax.ShapeDtypeStruct((1, 1), jnp.float32),
    )(x_sim, yidx, idcg)

    return total[0, 0] / n
